# Initial kernel scaffold; baseline (speedup 1.0000x reference)
import jax
import jax.numpy as jnp
from jax.experimental import pallas as pl


def kernel(user_ids, item_ids, user_emb, item_emb, W1, b1, W2, b2, W3, b3):
    raise NotImplementedError("write your pallas kernel here")


# speedup vs baseline: 2.4814x; 2.4814x over previous
import functools

import jax
import jax.numpy as jnp
from jax import lax
from jax.experimental import pallas as pl
from jax.experimental.pallas import tpu as pltpu
from jax.experimental.pallas import tpu_sc as plsc

_NC = 2
_NS = 16
_NW = _NC * _NS
_CHUNK = 128


def _sc_gather(uids3, iids3, user_emb, item_emb, interpret=False):
    nw, nchunk, chunk = uids3.shape
    d = user_emb.shape[1]
    rows_per_w = nchunk * chunk
    b = nw * rows_per_w
    mesh = plsc.VectorSubcoreMesh(core_axis_name="c", subcore_axis_name="s")

    @functools.partial(
        pl.kernel,
        out_type=(
            jax.ShapeDtypeStruct((b, d), jnp.float32),
            jax.ShapeDtypeStruct((b, d), jnp.float32),
        ),
        mesh=mesh,
        scratch_types=[
            pltpu.VMEM((nchunk, chunk), jnp.int32),
            pltpu.VMEM((rows_per_w, d), jnp.float32),
            pltpu.SemaphoreType.DMA,
        ],
        interpret=interpret,
    )
    def gather_kernel(u_hbm, i_hbm, ue_hbm, ie_hbm, out_u, out_i,
                      idx_v, rows_v, sem):
        wid = lax.axis_index("s") * _NC + lax.axis_index("c")
        base = wid * rows_per_w
        for ids_hbm, table_hbm, out_hbm in (
            (u_hbm, ue_hbm, out_u),
            (i_hbm, ie_hbm, out_i),
        ):
            pltpu.sync_copy(ids_hbm.at[wid], idx_v)
            handles = [
                pltpu.async_copy(
                    table_hbm.at[idx_v.at[j]],
                    rows_v.at[pl.ds(j * chunk, chunk)],
                    sem,
                )
                for j in range(nchunk)
            ]
            for h in handles:
                h.wait()
            pltpu.sync_copy(rows_v, out_hbm.at[pl.ds(base, rows_per_w)])

    return gather_kernel(uids3, iids3, user_emb, item_emb)


def _mlp_body(u_ref, i_ref, w1u_ref, w1i_ref, b1_ref, w2_ref, b2_ref,
              w3_ref, b3_ref, o_ref):
    h = jnp.dot(u_ref[...], w1u_ref[...], preferred_element_type=jnp.float32)
    h = h + jnp.dot(i_ref[...], w1i_ref[...],
                    preferred_element_type=jnp.float32)
    h = jnp.maximum(h + b1_ref[...], 0.0)
    h2 = jnp.dot(h, w2_ref[...], preferred_element_type=jnp.float32)
    h2 = jnp.maximum(h2 + b2_ref[...], 0.0)
    o_ref[...] = jnp.sum(h2 * w3_ref[...], axis=1) + b3_ref[0]


def _tc_mlp(ug, ig, W1, b1, W2, b2, W3, b3, interpret=False):
    b, d = ug.shape
    tile = 2048
    w1u = W1[:, :d].T
    w1i = W1[:, d:].T
    w2 = W2.T
    b1r = b1.reshape(1, -1)
    b2r = b2.reshape(1, -1)
    grid = (b // tile,)
    full = lambda shape: pl.BlockSpec(shape, lambda i: (0,) * len(shape))
    return pl.pallas_call(
        _mlp_body,
        grid=grid,
        in_specs=[
            pl.BlockSpec((tile, d), lambda i: (i, 0)),
            pl.BlockSpec((tile, d), lambda i: (i, 0)),
            full(w1u.shape),
            full(w1i.shape),
            full(b1r.shape),
            full(w2.shape),
            full(b2r.shape),
            full(W3.shape),
            pl.BlockSpec(memory_space=pltpu.SMEM),
        ],
        out_specs=pl.BlockSpec((tile,), lambda i: (i,)),
        out_shape=jax.ShapeDtypeStruct((b,), jnp.float32),
        interpret=interpret,
    )(ug, ig, w1u, w1i, b1r, w2, b2r, W3, b3)


def kernel(user_ids, item_ids, user_emb, item_emb, W1, b1, W2, b2, W3, b3):
    b = user_ids.shape[0]
    rows_per_w = b // _NW
    nchunk = rows_per_w // _CHUNK
    uids3 = user_ids.astype(jnp.int32).reshape(_NW, nchunk, _CHUNK)
    iids3 = item_ids.astype(jnp.int32).reshape(_NW, nchunk, _CHUNK)
    ug, ig = _sc_gather(uids3, iids3, user_emb, item_emb)
    return _tc_mlp(ug, ig, W1, b1, W2, b2, W3, b3)


# speedup vs baseline: 3.3675x; 1.3571x over previous
import functools

import jax
import jax.numpy as jnp
from jax import lax
from jax.experimental import pallas as pl
from jax.experimental.pallas import tpu as pltpu
from jax.experimental.pallas import tpu_sc as plsc

_NC = 2
_NS = 16
_NW = _NC * _NS
_CHUNK = 128


def _sc_gather(uids3, iids3, user_emb, item_emb, interpret=False):
    nw, nchunk, chunk = uids3.shape
    d = user_emb.shape[1]
    rows_per_w = nchunk * chunk
    b = nw * rows_per_w
    mesh = plsc.VectorSubcoreMesh(core_axis_name="c", subcore_axis_name="s")

    @functools.partial(
        pl.kernel,
        out_type=(
            jax.ShapeDtypeStruct((b, d), jnp.float32),
            jax.ShapeDtypeStruct((b, d), jnp.float32),
        ),
        mesh=mesh,
        scratch_types=[
            pltpu.VMEM((nchunk, chunk), jnp.int32),
            pltpu.VMEM((rows_per_w, d), jnp.float32),
            pltpu.SemaphoreType.DMA,
        ],
        interpret=interpret,
    )
    def gather_kernel(u_hbm, i_hbm, ue_hbm, ie_hbm, out_u, out_i,
                      idx_v, rows_v, sem):
        wid = lax.axis_index("s") * _NC + lax.axis_index("c")
        base = wid * rows_per_w
        for ids_hbm, table_hbm, out_hbm in (
            (u_hbm, ue_hbm, out_u),
            (i_hbm, ie_hbm, out_i),
        ):
            pltpu.sync_copy(ids_hbm.at[wid], idx_v)
            handles = [
                pltpu.async_copy(
                    table_hbm.at[idx_v.at[j]],
                    rows_v.at[pl.ds(j * chunk, chunk)],
                    sem,
                )
                for j in range(nchunk)
            ]
            for h in handles:
                h.wait()
            pltpu.sync_copy(rows_v, out_hbm.at[pl.ds(base, rows_per_w)])

    return gather_kernel(uids3, iids3, user_emb, item_emb)


def _mlp_body(u_ref, i_ref, w1u_ref, w1i_ref, b1_ref, w2_ref, b2_ref,
              w3_ref, b3_ref, o_ref):
    h = jnp.dot(u_ref[...], w1u_ref[...], preferred_element_type=jnp.float32)
    h = h + jnp.dot(i_ref[...], w1i_ref[...],
                    preferred_element_type=jnp.float32)
    h = jnp.maximum(h + b1_ref[...], 0.0)
    h2 = jnp.dot(h, w2_ref[...], preferred_element_type=jnp.float32)
    h2 = jnp.maximum(h2 + b2_ref[...], 0.0)
    z = lax.dot_general(w3_ref[...], h2, (((1,), (1,)), ((), ())),
                        preferred_element_type=jnp.float32)
    o_ref[...] = z.reshape(o_ref.shape) + b3_ref[0]


def _tc_mlp(ug, ig, W1, b1, W2, b2, W3, b3, interpret=False):
    b, d = ug.shape
    tile = 2048
    w1u = W1[:, :d].T
    w1i = W1[:, d:].T
    w2 = W2.T
    b1r = b1.reshape(1, -1)
    b2r = b2.reshape(1, -1)
    grid = (b // tile,)
    full = lambda shape: pl.BlockSpec(shape, lambda i: (0,) * len(shape))
    return pl.pallas_call(
        _mlp_body,
        grid=grid,
        in_specs=[
            pl.BlockSpec((tile, d), lambda i: (i, 0)),
            pl.BlockSpec((tile, d), lambda i: (i, 0)),
            full(w1u.shape),
            full(w1i.shape),
            full(b1r.shape),
            full(w2.shape),
            full(b2r.shape),
            full(W3.shape),
            pl.BlockSpec(memory_space=pltpu.SMEM),
        ],
        out_specs=pl.BlockSpec((tile,), lambda i: (i,)),
        out_shape=jax.ShapeDtypeStruct((b,), jnp.float32),
        interpret=interpret,
    )(ug, ig, w1u, w1i, b1r, w2, b2r, W3, b3)


def kernel(user_ids, item_ids, user_emb, item_emb, W1, b1, W2, b2, W3, b3):
    b = user_ids.shape[0]
    rows_per_w = b // _NW
    nchunk = rows_per_w // _CHUNK
    uids3 = user_ids.astype(jnp.int32).reshape(_NW, nchunk, _CHUNK)
    iids3 = item_ids.astype(jnp.int32).reshape(_NW, nchunk, _CHUNK)
    ug, ig = _sc_gather(uids3, iids3, user_emb, item_emb)
    return _tc_mlp(ug, ig, W1, b1, W2, b2, W3, b3)
